# Initial kernel scaffold; baseline (speedup 1.0000x reference)
import jax
import jax.numpy as jnp
from jax.experimental import pallas as pl


def kernel(x, ln_g, ln_b, gate_W, cp_W1, cp_b1, cp_W2, cp_b2, fc1s, b1s, fc2s, b2s):
    raise NotImplementedError("write your pallas kernel here")


# speedup vs baseline: 1.5615x; 1.5615x over previous
import functools

import jax
import jax.numpy as jnp
from jax import lax
from jax.experimental import pallas as pl
from jax.experimental.pallas import tpu as pltpu

BS, D, E, DD, K = 8192, 1024, 8, 4096, 1024
_IT = False


def _ln_gate_body(x_ref, g_ref, b_ref, gw_ref, nx_ref, st_ref):
    xb = x_ref[...]
    mu = jnp.mean(xb, axis=1, keepdims=True)
    var = jnp.mean((xb - mu) ** 2, axis=1, keepdims=True)
    nx = (xb - mu) * lax.rsqrt(var + 1e-5) * g_ref[...][None, :] + b_ref[...][None, :]
    nx_ref[...] = nx
    sc = lax.dot_general(gw_ref[...], nx, (((1,), (1,)), ((), ())),
                         preferred_element_type=jnp.float32)
    st_ref[...] = (jnp.tanh(sc) + 1.0) * 0.5


def _ln_gate(xf, ln_g, ln_b, gate_W):
    bt = 512
    grid = (BS // bt,)
    return pl.pallas_call(
        _ln_gate_body,
        grid=grid,
        in_specs=[
            pl.BlockSpec((bt, D), lambda i: (i, 0)),
            pl.BlockSpec((D,), lambda i: (0,)),
            pl.BlockSpec((D,), lambda i: (0,)),
            pl.BlockSpec((E, D), lambda i: (0, 0)),
        ],
        out_specs=[
            pl.BlockSpec((bt, D), lambda i: (i, 0)),
            pl.BlockSpec((E, bt), lambda i: (0, i)),
        ],
        out_shape=[
            jax.ShapeDtypeStruct((BS, D), jnp.float32),
            jax.ShapeDtypeStruct((E, BS), jnp.float32),
        ],
        interpret=_IT,
    )(xf, ln_g, ln_b, gate_W)


def _cp_body(nx_ref, mT_ref, w1_ref, b1_ref, w2_ref, b2_ref, acc_ref):
    i = pl.program_id(0)
    nxb = nx_ref[...].astype(jnp.bfloat16)
    w1 = w1_ref[...].astype(jnp.bfloat16)
    h = lax.dot_general(nxb, w1, (((1,), (1,)), ((), ())),
                        preferred_element_type=jnp.float32)
    h = jax.nn.gelu(h + b1_ref[...][None, :], approximate=True)
    lg = lax.dot_general(w2_ref[...].astype(jnp.bfloat16), h.astype(jnp.bfloat16),
                         (((1,), (1,)), ((), ())),
                         preferred_element_type=jnp.float32)
    lg = lg + b2_ref[...][:, None]
    m = mT_ref[...]
    bce = jnp.maximum(lg, 0.0) - lg * m + jnp.log1p(jnp.exp(-jnp.abs(lg)))
    s = jnp.sum(bce)
    lane = lax.broadcasted_iota(jnp.int32, (1, 128), 1)
    sv = jnp.where(lane == 0, s, 0.0)

    @pl.when(i == 0)
    def _():
        acc_ref[...] = jnp.zeros_like(acc_ref)

    acc_ref[...] += sv


def _cp_loss(nx, maskT, cp_W1, cp_b1, cp_W2, cp_b2):
    bt = 512
    grid = (BS // bt,)
    out = pl.pallas_call(
        _cp_body,
        grid=grid,
        in_specs=[
            pl.BlockSpec((bt, D), lambda i: (i, 0)),
            pl.BlockSpec((E, bt), lambda i: (0, i)),
            pl.BlockSpec((D, D), lambda i: (0, 0)),
            pl.BlockSpec((D,), lambda i: (0,)),
            pl.BlockSpec((E, D), lambda i: (0, 0)),
            pl.BlockSpec((E,), lambda i: (0,)),
        ],
        out_specs=pl.BlockSpec((1, 128), lambda i: (0, 0)),
        out_shape=jax.ShapeDtypeStruct((1, 128), jnp.float32),
        interpret=_IT,
    )(nx, maskT, cp_W1, cp_b1, cp_W2, cp_b2)
    return out[0, 0] / (BS * E)


def _moe_body(xg_ref, fc1_ref, b1_ref, fc2_ref, b2_ref, ks_ref, y_ref, acc_ref):
    dd = pl.program_id(1)
    nd = pl.num_programs(1)
    xb = xg_ref[0].astype(jnp.bfloat16)
    w1 = fc1_ref[0].astype(jnp.bfloat16)
    h = lax.dot_general(xb, w1, (((1,), (1,)), ((), ())),
                        preferred_element_type=jnp.float32)
    h = jax.nn.gelu(h + b1_ref[0, 0][None, :], approximate=True).astype(jnp.bfloat16)
    w2 = fc2_ref[0].astype(jnp.bfloat16)
    yp = lax.dot_general(h, w2, (((1,), (1,)), ((), ())),
                         preferred_element_type=jnp.float32)

    @pl.when(dd == 0)
    def _():
        acc_ref[...] = jnp.zeros_like(acc_ref)

    acc_ref[...] += yp

    @pl.when(dd == nd - 1)
    def _():
        y_ref[0] = (acc_ref[...] + b2_ref[0, 0][None, :]) * ks_ref[0, 0][:, None]


def _moe_mlp(xg, fc1s, b1s, fc2s, b2s, kscores):
    ddb = 1024
    grid = (E, DD // ddb)
    return pl.pallas_call(
        _moe_body,
        grid=grid,
        in_specs=[
            pl.BlockSpec((1, K, D), lambda e, d: (e, 0, 0)),
            pl.BlockSpec((1, ddb, D), lambda e, d: (e, d, 0)),
            pl.BlockSpec((1, 1, ddb), lambda e, d: (e * (DD // ddb) + d, 0, 0)),
            pl.BlockSpec((1, D, ddb), lambda e, d: (e, 0, d)),
            pl.BlockSpec((1, 1, D), lambda e, d: (e, 0, 0)),
            pl.BlockSpec((1, 1, K), lambda e, d: (e, 0, 0)),
        ],
        out_specs=pl.BlockSpec((1, K, D), lambda e, d: (e, 0, 0)),
        out_shape=jax.ShapeDtypeStruct((E, K, D), jnp.float32),
        scratch_shapes=[pltpu.VMEM((K, D), jnp.float32)],
        interpret=_IT,
    )(xg, fc1s, b1s.reshape(E * (DD // ddb), 1, ddb), fc2s,
      b2s.reshape(E, 1, D), kscores.reshape(E, 1, K))


def kernel(x, ln_g, ln_b, gate_W, cp_W1, cp_b1, cp_W2, cp_b2, fc1s, b1s, fc2s, b2s):
    xf = x.reshape(BS, D)
    nx, scoresT = _ln_gate(xf, ln_g, ln_b, gate_W)

    _, top_idx = lax.top_k(scoresT, K)
    idxs = jnp.sort(top_idx, axis=1).astype(jnp.int32)
    kscores = jnp.take_along_axis(scoresT, idxs, axis=1)
    maskT = jnp.zeros((E, BS), jnp.float32).at[
        jnp.arange(E)[:, None], idxs].set(1.0)

    loss = _cp_loss(nx, maskT, cp_W1, cp_b1, cp_W2, cp_b2)

    xg = nx[idxs.reshape(-1)].reshape(E, K, D)

    y = _moe_mlp(xg, fc1s, b1s, fc2s, b2s, kscores)

    out = xf.at[idxs.reshape(-1)].add(y.reshape(E * K, D))
    return out.reshape(x.shape), loss
